# Initial kernel scaffold; baseline (speedup 1.0000x reference)
import jax
import jax.numpy as jnp
from jax.experimental import pallas as pl


def kernel(x, W1, b1, W2, b2, Wg, bg):
    raise NotImplementedError("write your pallas kernel here")


# speedup vs baseline: 2.7042x; 2.7042x over previous
import jax
import jax.numpy as jnp
from jax.experimental import pallas as pl
from jax.experimental.pallas import tpu as pltpu

E = 8
TOP_K = 2
DIM = 768
DFF = DIM * 4
T = 2048
BJ = 512
J = DFF // BJ


def _moe_body(x_ref, W1_ref, b1_ref, W2_ref, b2_ref, Wg_ref, bg_ref,
              out_ref, xb_ref, w_ref):
    e = pl.program_id(0)
    j = pl.program_id(1)

    @pl.when((e == 0) & (j == 0))
    def _init():
        x = x_ref[...]
        xb_ref[...] = x.astype(jnp.bfloat16)
        logits = jnp.dot(x, Wg_ref[...],
                         preferred_element_type=jnp.float32) + bg_ref[...]
        m1 = jnp.full((T, 1), -jnp.inf, jnp.float32)
        a1 = jnp.zeros((T, 1), jnp.int32)
        for k in range(E):
            lk = logits[:, k:k + 1]
            better = lk > m1
            a1 = jnp.where(better, k, a1)
            m1 = jnp.where(better, lk, m1)
        m2 = jnp.full((T, 1), -jnp.inf, jnp.float32)
        a2 = jnp.zeros((T, 1), jnp.int32)
        for k in range(E):
            lk = logits[:, k:k + 1]
            better = (lk > m2) & (a1 != k)
            a2 = jnp.where(better, k, a2)
            m2 = jnp.where(better, lk, m2)
        e2 = jnp.exp(m2 - m1)
        denom = 1.0 + e2
        w1 = 1.0 / denom
        w2 = e2 / denom
        iota = jax.lax.broadcasted_iota(jnp.int32, (T, E), 1)
        w_ref[...] = (jnp.where(iota == a1, w1, 0.0)
                      + jnp.where(iota == a2, w2, 0.0))
        out_ref[...] = jnp.zeros_like(out_ref)

    iota_e = jax.lax.broadcasted_iota(jnp.int32, (T, E), 1)
    wcol = jnp.sum(jnp.where(iota_e == e, w_ref[...], 0.0),
                   axis=1, keepdims=True)

    h = jnp.dot(xb_ref[...], W1_ref[0].astype(jnp.bfloat16),
                preferred_element_type=jnp.float32) + b1_ref[0]
    h = (h * 0.5 * (1.0 + jax.lax.erf(h * 0.7071067811865476))).astype(jnp.bfloat16)
    part = jnp.dot(h, W2_ref[0].astype(jnp.bfloat16),
                   preferred_element_type=jnp.float32)

    @pl.when(j == 0)
    def _bias2():
        out_ref[...] += wcol * b2_ref[0]

    out_ref[...] += part * wcol


def kernel(x, W1, b1, W2, b2, Wg, bg):
    B, S, _ = x.shape
    x2 = x.reshape(S, DIM)
    bg2 = bg.reshape(1, E)
    b1r = b1.reshape(E, 1, DFF)
    b2r = b2.reshape(E, 1, DIM)

    out = pl.pallas_call(
        _moe_body,
        grid=(E, J),
        in_specs=[
            pl.BlockSpec((T, DIM), lambda e, j: (0, 0)),
            pl.BlockSpec((1, DIM, BJ), lambda e, j: (e, 0, j)),
            pl.BlockSpec((1, 1, BJ), lambda e, j: (e, 0, j)),
            pl.BlockSpec((1, BJ, DIM), lambda e, j: (e, j, 0)),
            pl.BlockSpec((1, 1, DIM), lambda e, j: (e, 0, 0)),
            pl.BlockSpec((DIM, E), lambda e, j: (0, 0)),
            pl.BlockSpec((1, E), lambda e, j: (0, 0)),
        ],
        out_specs=pl.BlockSpec((T, DIM), lambda e, j: (0, 0)),
        out_shape=jax.ShapeDtypeStruct((T, DIM), jnp.float32),
        scratch_shapes=[
            pltpu.VMEM((T, DIM), jnp.bfloat16),
            pltpu.VMEM((T, E), jnp.float32),
        ],
        compiler_params=pltpu.CompilerParams(
            dimension_semantics=("arbitrary", "arbitrary"),
        ),
    )(x2, W1, b1r, W2, b2r, Wg, bg2)
    return out.reshape(B, S, DIM)
